# baseline (speedup 1.0000x reference)
import functools

import jax
import jax.numpy as jnp
from jax import lax
from jax.experimental import pallas as pl
from jax.experimental.pallas import tpu as pltpu
from jax.experimental.pallas import tpu_sc as plsc

HIDDEN = 64
GAMMA = 12.0
BATCH = 16384

NC = 2
NS = 16
NW = NC * NS
BPW = BATCH // NW
CH = 128
NCHUNK = BPW // CH
SBLK = 16
PAD = SBLK + 1

_mesh = plsc.VectorSubcoreMesh(core_axis_name="c", subcore_axis_name="s")


@functools.partial(
    pl.kernel,
    out_type=jax.ShapeDtypeStruct((BATCH,), jnp.float32),
    mesh=_mesh,
    compiler_params=pltpu.CompilerParams(needs_layout_passes=False),
    scratch_types=[
        pltpu.VMEM((BPW,), jnp.int32),
        pltpu.VMEM((BPW,), jnp.int32),
        pltpu.VMEM((BPW,), jnp.int32),
        pltpu.VMEM((CH, HIDDEN), jnp.float32),
        pltpu.VMEM((CH, HIDDEN), jnp.float32),
        pltpu.VMEM((CH, HIDDEN), jnp.float32),
        pltpu.VMEM((BPW,), jnp.float32),
        pltpu.VMEM((SBLK * PAD,), jnp.float32),
        pltpu.SemaphoreType.DMA,
        pltpu.SemaphoreType.DMA,
        pltpu.SemaphoreType.DMA,
        pltpu.SemaphoreType.DMA,
    ],
)
def _transe_score(hidx_hbm, ridx_hbm, tidx_hbm, ent_hbm, rel_hbm, out_hbm,
                  hidx_v, ridx_v, tidx_v, h_v, r_v, t_v, out_v, scr_v,
                  sem0, sem1, sem2, sem3):
    sems = (sem0, sem1, sem2, sem3)
    wid = lax.axis_index("s") * NC + lax.axis_index("c")
    base = wid * BPW

    pltpu.sync_copy(hidx_hbm.at[pl.ds(base, BPW)], hidx_v)
    pltpu.sync_copy(ridx_hbm.at[pl.ds(base, BPW)], ridx_v)
    pltpu.sync_copy(tidx_hbm.at[pl.ds(base, BPW)], tidx_v)

    lane = lax.broadcasted_iota(jnp.int32, (SBLK,), 0)
    gamma = jnp.full((SBLK,), GAMMA, jnp.float32)

    def chunk(c, carry):
        cbase = c * CH
        cps = []
        for j in range(CH // SBLK):
            sl = pl.ds(cbase + j * SBLK, SBLK)
            hvec = hidx_v[sl]
            rvec = ridx_v[sl]
            tvec = tidx_v[sl]
            for k in range(SBLK):
                kk = j * SBLK + k
                cps.append(pltpu.async_copy(ent_hbm.at[hvec[k]], h_v.at[kk],
                                            sems[(3 * kk) % 4]))
                cps.append(pltpu.async_copy(rel_hbm.at[rvec[k]], r_v.at[kk],
                                            sems[(3 * kk + 1) % 4]))
                cps.append(pltpu.async_copy(ent_hbm.at[tvec[k]], t_v.at[kk],
                                            sems[(3 * kk + 2) % 4]))
        for cp in cps:
            cp.wait()

        for blk in range(CH // SBLK):
            for k in range(SBLK):
                kk = blk * SBLK + k
                acc = None
                for ci in range(HIDDEN // 16):
                    sl = pl.ds(ci * 16, 16)
                    d = jnp.abs(h_v[kk, sl] + r_v[kk, sl] - t_v[kk, sl])
                    acc = d if acc is None else acc + d
                scr_v[pl.ds(k * PAD, SBLK)] = acc
            tot = plsc.load_gather(scr_v, [lane * PAD])
            for rr in range(1, SBLK):
                tot = tot + plsc.load_gather(scr_v, [lane * PAD + rr])
            out_v[pl.ds(cbase + blk * SBLK, SBLK)] = gamma - tot
        return carry

    lax.fori_loop(0, NCHUNK, chunk, 0)
    pltpu.sync_copy(out_v, out_hbm.at[pl.ds(base, BPW)])


@jax.jit
def kernel(sample, entity_embedding, relation_embedding):
    score = _transe_score(
        sample[:, 0], sample[:, 1], sample[:, 2],
        entity_embedding, relation_embedding)
    return score.reshape(BATCH, 1)


# speedup vs baseline: 1.0403x; 1.0403x over previous
import functools

import jax
import jax.numpy as jnp
from jax import lax
from jax.experimental import pallas as pl
from jax.experimental.pallas import tpu as pltpu
from jax.experimental.pallas import tpu_sc as plsc
from jax.experimental.layout import Format, Layout, with_layout_constraint

HIDDEN = 64
GAMMA = 12.0
BATCH = 16384

NC = 2
NS = 16
NW = NC * NS
BPW = BATCH // NW
IDX_CHUNK = 128
NCHUNK = BPW // IDX_CHUNK
SBLK = 16
PAD = SBLK + 1

_mesh = plsc.VectorSubcoreMesh(core_axis_name="c", subcore_axis_name="s")


@functools.partial(
    pl.kernel,
    out_type=jax.ShapeDtypeStruct((NW, BPW), jnp.float32),
    mesh=_mesh,
    compiler_params=pltpu.CompilerParams(
        needs_layout_passes=False, use_tc_tiling_on_sc=False),
    scratch_types=[
        pltpu.VMEM((NCHUNK, IDX_CHUNK), jnp.int32),
        pltpu.VMEM((NCHUNK, IDX_CHUNK), jnp.int32),
        pltpu.VMEM((NCHUNK, IDX_CHUNK), jnp.int32),
        pltpu.VMEM((BPW, HIDDEN), jnp.float32),
        pltpu.VMEM((BPW, HIDDEN), jnp.float32),
        pltpu.VMEM((BPW, HIDDEN), jnp.float32),
        pltpu.VMEM((BPW,), jnp.float32),
        pltpu.VMEM((SBLK * PAD,), jnp.float32),
        pltpu.SemaphoreType.DMA,
    ],
)
def _transe_score(hidx_hbm, ridx_hbm, tidx_hbm, ent_hbm, rel_hbm, out_hbm,
                  hidx_v, ridx_v, tidx_v, h_v, r_v, t_v, out_v, scr_v, sem):
    wid = lax.axis_index("s") * NC + lax.axis_index("c")

    pltpu.sync_copy(hidx_hbm.at[wid], hidx_v)
    pltpu.sync_copy(ridx_hbm.at[wid], ridx_v)
    pltpu.sync_copy(tidx_hbm.at[wid], tidx_v)

    copies = []
    for j in range(NCHUNK):
        rows = pl.ds(j * IDX_CHUNK, IDX_CHUNK)
        copies.append(pltpu.async_copy(ent_hbm.at[hidx_v.at[j]], h_v.at[rows, :], sem))
        copies.append(pltpu.async_copy(rel_hbm.at[ridx_v.at[j]], r_v.at[rows, :], sem))
        copies.append(pltpu.async_copy(ent_hbm.at[tidx_v.at[j]], t_v.at[rows, :], sem))
    for cp in copies:
        cp.wait()

    lane = lax.broadcasted_iota(jnp.int32, (SBLK,), 0)
    gamma = jnp.full((SBLK,), GAMMA, jnp.float32)

    def block(b, carry):
        base = b * SBLK
        for k in range(SBLK):
            i = base + k
            acc = None
            for c in range(HIDDEN // 16):
                sl = pl.ds(c * 16, 16)
                d = jnp.abs(h_v[i, sl] + r_v[i, sl] - t_v[i, sl])
                acc = d if acc is None else acc + d
            scr_v[pl.ds(k * PAD, SBLK)] = acc
        tot = plsc.load_gather(scr_v, [lane * PAD])
        for rr in range(1, SBLK):
            tot = tot + plsc.load_gather(scr_v, [lane * PAD + rr])
        out_v[pl.ds(base, SBLK)] = gamma - tot
        return carry

    lax.fori_loop(0, BPW // SBLK, block, 0)
    pltpu.sync_copy(out_v, out_hbm.at[wid])


@jax.jit
def kernel(sample, entity_embedding, relation_embedding):
    hidx = sample[:, 0].reshape(NW, NCHUNK, IDX_CHUNK)
    ridx = sample[:, 1].reshape(NW, NCHUNK, IDX_CHUNK)
    tidx = sample[:, 2].reshape(NW, NCHUNK, IDX_CHUNK)
    t8 = Layout(major_to_minor=(0, 1), tiling=((8,),))
    ent = with_layout_constraint(entity_embedding, t8)
    rel = with_layout_constraint(relation_embedding, t8)
    score = _transe_score(hidx, ridx, tidx, ent, rel)
    return score.reshape(BATCH, 1)
